# baseline (speedup 1.0000x reference)
import jax
import jax.numpy as jnp
from jax.experimental import pallas as pl
from jax.experimental.pallas import tpu as pltpu

_S = 64
_BIG = 1e30
_BLOCK = 65536


def _scene_body(pd_ref, a2_ref, tab_ref, outf_ref, outi_ref):
    pd = pd_ref[...]

    y = jnp.dot(a2_ref[...], pd)
    pn = y[0:_S, :]
    dn = y[_S:2 * _S, :]

    off = a2_ref[0:_S, 7:8]
    ok = jnp.abs(dn) > 1e-8
    t0 = (off - pn) / dn
    valid = ok & (t0 > 1e-6)
    t = jnp.where(valid, t0, _BIG)

    min_t = jnp.min(t, axis=0, keepdims=True)
    iota = jax.lax.broadcasted_iota(jnp.int32, (_S, 1), 0)
    idx = jnp.min(jnp.where(t == min_t, iota, _S), axis=0, keepdims=True)

    h = jnp.where(iota == idx, 1.0, 0.0)
    g = jax.lax.dot(tab_ref[...], h,
                    precision=jax.lax.Precision.HIGHEST)
    nwx = g[0:1, :]
    nwy = g[1:2, :]
    nwz = g[2:3, :]
    rw = g[4:5, :]
    ew = g[5:6, :]
    sw = g[6:7, :]

    px = pd[0:1, :]
    py = pd[1:2, :]
    pz = pd[2:3, :]
    dx = pd[3:4, :]
    dy = pd[4:5, :]
    dz = pd[5:6, :]
    inten = pd[6:7, :]

    active = (min_t < _BIG) & (inten > 0.0)
    dnw = dx * nwx + dy * nwy + dz * nwz
    two_dnw = 2.0 * dnw

    outf_ref[0:1, :] = jnp.where(active, px + min_t * dx, px)
    outf_ref[1:2, :] = jnp.where(active, py + min_t * dy, py)
    outf_ref[2:3, :] = jnp.where(active, pz + min_t * dz, pz)
    outf_ref[3:4, :] = jnp.where(active, dx - two_dnw * nwx, dx)
    outf_ref[4:5, :] = jnp.where(active, dy - two_dnw * nwy, dy)
    outf_ref[5:6, :] = jnp.where(active, dz - two_dnw * nwz, dz)
    outf_ref[6:7, :] = jnp.where(active, inten * rw, inten)
    outf_ref[7:8, :] = jnp.zeros_like(inten)

    outi_ref[0:1, :] = (ew + 0.5).astype(jnp.int32)
    outi_ref[1:2, :] = (sw + 0.5).astype(jnp.int32)


def kernel(pos, dir, intensity, normals, offsets, reflectivity,
           map_to_element, map_to_surface):
    n = pos.shape[0]
    block = _BLOCK if n % _BLOCK == 0 else n
    grid = n // block

    pd = jnp.concatenate([
        pos.T, dir.T, intensity[None, :],
        jnp.zeros((1, n), jnp.float32),
    ], axis=0)

    s = normals.shape[0]
    a2 = jnp.zeros((2 * s, 8), jnp.float32)
    a2 = a2.at[0:s, 0:3].set(normals)
    a2 = a2.at[s:2 * s, 3:6].set(normals)
    a2 = a2.at[0:s, 7].set(offsets)

    tab = jnp.concatenate([
        normals.T,
        offsets[None, :],
        reflectivity[None, :],
        map_to_element[None, :].astype(jnp.float32),
        map_to_surface[None, :].astype(jnp.float32),
        jnp.zeros((1, s), jnp.float32),
    ], axis=0)

    outf, outi = pl.pallas_call(
        _scene_body,
        grid=(grid,),
        in_specs=[
            pl.BlockSpec((8, block), lambda i: (0, i)),
            pl.BlockSpec((2 * _S, 8), lambda i: (0, 0)),
            pl.BlockSpec((8, _S), lambda i: (0, 0)),
        ],
        out_specs=(
            pl.BlockSpec((8, block), lambda i: (0, i)),
            pl.BlockSpec((2, block), lambda i: (0, i)),
        ),
        out_shape=(
            jax.ShapeDtypeStruct((8, n), jnp.float32),
            jax.ShapeDtypeStruct((2, n), jnp.int32),
        ),
        compiler_params=pltpu.CompilerParams(
            dimension_semantics=("parallel",)),
    )(pd, a2, tab)

    next_pos = outf[0:3, :].T
    next_dir = outf[3:6, :].T
    next_intensity = outf[6, :]
    return (next_pos, next_dir, next_intensity, outi[0, :], outi[1, :])


# speedup vs baseline: 1.0054x; 1.0054x over previous
import jax
import jax.numpy as jnp
from jax.experimental import pallas as pl
from jax.experimental.pallas import tpu as pltpu

_S = 64
_BIG = 1e30
_BLOCK = 32768


_CHUNKS = 4


def _scene_body(pd_ref, a2_ref, tab_ref, outf_ref, outi_ref):
    b = pd_ref.shape[1]
    c = b // _CHUNKS
    for i in range(_CHUNKS):
        sl = pl.ds(i * c, c)
        _scene_chunk(pd_ref[:, sl], a2_ref, tab_ref, outf_ref, outi_ref, sl)


def _scene_chunk(pd, a2_ref, tab_ref, outf_ref, outi_ref, sl):
    y = jnp.dot(a2_ref[...], pd)
    pn = y[0:_S, :]
    dn = y[_S:2 * _S, :]

    off = a2_ref[0:_S, 7:8]
    ok = jnp.abs(dn) > 1e-8
    t0 = (off - pn) / dn
    valid = ok & (t0 > 1e-6)
    t = jnp.where(valid, t0, _BIG)

    min_t = jnp.min(t, axis=0, keepdims=True)
    iota = jax.lax.broadcasted_iota(jnp.int32, (_S, 1), 0)
    idx = jnp.min(jnp.where(t == min_t, iota, _S), axis=0, keepdims=True)

    h = jnp.where(iota == idx, 1.0, 0.0)
    g = jax.lax.dot(tab_ref[...], h,
                    precision=jax.lax.Precision.HIGHEST)
    nwx = g[0:1, :]
    nwy = g[1:2, :]
    nwz = g[2:3, :]
    rw = g[4:5, :]
    ew = g[5:6, :]
    sw = g[6:7, :]

    px = pd[0:1, :]
    py = pd[1:2, :]
    pz = pd[2:3, :]
    dx = pd[3:4, :]
    dy = pd[4:5, :]
    dz = pd[5:6, :]
    inten = pd[6:7, :]

    active = (min_t < _BIG) & (inten > 0.0)
    dnw = dx * nwx + dy * nwy + dz * nwz
    two_dnw = 2.0 * dnw

    outf_ref[0:1, sl] = jnp.where(active, px + min_t * dx, px)
    outf_ref[1:2, sl] = jnp.where(active, py + min_t * dy, py)
    outf_ref[2:3, sl] = jnp.where(active, pz + min_t * dz, pz)
    outf_ref[3:4, sl] = jnp.where(active, dx - two_dnw * nwx, dx)
    outf_ref[4:5, sl] = jnp.where(active, dy - two_dnw * nwy, dy)
    outf_ref[5:6, sl] = jnp.where(active, dz - two_dnw * nwz, dz)
    outf_ref[6:7, sl] = jnp.where(active, inten * rw, inten)
    outf_ref[7:8, sl] = jnp.zeros_like(inten)

    outi_ref[0:1, sl] = (ew + 0.5).astype(jnp.int32)
    outi_ref[1:2, sl] = (sw + 0.5).astype(jnp.int32)


def kernel(pos, dir, intensity, normals, offsets, reflectivity,
           map_to_element, map_to_surface):
    n = pos.shape[0]
    block = _BLOCK if n % _BLOCK == 0 else n
    grid = n // block

    pd = jnp.concatenate([
        pos.T, dir.T, intensity[None, :],
        jnp.zeros((1, n), jnp.float32),
    ], axis=0)

    s = normals.shape[0]
    a2 = jnp.zeros((2 * s, 8), jnp.float32)
    a2 = a2.at[0:s, 0:3].set(normals)
    a2 = a2.at[s:2 * s, 3:6].set(normals)
    a2 = a2.at[0:s, 7].set(offsets)

    tab = jnp.concatenate([
        normals.T,
        offsets[None, :],
        reflectivity[None, :],
        map_to_element[None, :].astype(jnp.float32),
        map_to_surface[None, :].astype(jnp.float32),
        jnp.zeros((1, s), jnp.float32),
    ], axis=0)

    outf, outi = pl.pallas_call(
        _scene_body,
        grid=(grid,),
        in_specs=[
            pl.BlockSpec((8, block), lambda i: (0, i)),
            pl.BlockSpec((2 * _S, 8), lambda i: (0, 0)),
            pl.BlockSpec((8, _S), lambda i: (0, 0)),
        ],
        out_specs=(
            pl.BlockSpec((8, block), lambda i: (0, i)),
            pl.BlockSpec((2, block), lambda i: (0, i)),
        ),
        out_shape=(
            jax.ShapeDtypeStruct((8, n), jnp.float32),
            jax.ShapeDtypeStruct((2, n), jnp.int32),
        ),
        compiler_params=pltpu.CompilerParams(
            dimension_semantics=("parallel",)),
    )(pd, a2, tab)

    next_pos = outf[0:3, :].T
    next_dir = outf[3:6, :].T
    next_intensity = outf[6, :]
    return (next_pos, next_dir, next_intensity, outi[0, :], outi[1, :])


# speedup vs baseline: 1.0100x; 1.0046x over previous
import jax
import jax.numpy as jnp
from jax.experimental import pallas as pl
from jax.experimental.pallas import tpu as pltpu

_S = 64
_BIG = 1e30
_BLOCK = 32768


def _scene_body(pd_ref, a2_ref, tab_ref, outf_ref, outi_ref):
    pd = pd_ref[...]

    y = jnp.dot(a2_ref[...], pd)
    pn = y[0:_S, :]
    dn = y[_S:2 * _S, :]

    off = a2_ref[0:_S, 7:8]
    ok = jnp.abs(dn) > 1e-8
    t0 = (off - pn) / dn
    valid = ok & (t0 > 1e-6)
    t = jnp.where(valid, t0, _BIG)

    min_t = jnp.min(t, axis=0, keepdims=True)
    iota = jax.lax.broadcasted_iota(jnp.int32, (_S, 1), 0)
    idx = jnp.min(jnp.where(t == min_t, iota, _S), axis=0, keepdims=True)

    h = jnp.where(iota == idx, 1.0, 0.0)
    g = jax.lax.dot(tab_ref[...], h,
                    precision=jax.lax.Precision.HIGHEST)
    nwx = g[0:1, :]
    nwy = g[1:2, :]
    nwz = g[2:3, :]
    rw = g[4:5, :]
    ew = g[5:6, :]
    sw = g[6:7, :]

    px = pd[0:1, :]
    py = pd[1:2, :]
    pz = pd[2:3, :]
    dx = pd[3:4, :]
    dy = pd[4:5, :]
    dz = pd[5:6, :]
    inten = pd[6:7, :]

    active = (min_t < _BIG) & (inten > 0.0)
    dnw = dx * nwx + dy * nwy + dz * nwz
    two_dnw = 2.0 * dnw

    outf_ref[0:1, :] = jnp.where(active, px + min_t * dx, px)
    outf_ref[1:2, :] = jnp.where(active, py + min_t * dy, py)
    outf_ref[2:3, :] = jnp.where(active, pz + min_t * dz, pz)
    outf_ref[3:4, :] = jnp.where(active, dx - two_dnw * nwx, dx)
    outf_ref[4:5, :] = jnp.where(active, dy - two_dnw * nwy, dy)
    outf_ref[5:6, :] = jnp.where(active, dz - two_dnw * nwz, dz)
    outf_ref[6:7, :] = jnp.where(active, inten * rw, inten)
    outf_ref[7:8, :] = jnp.zeros_like(inten)

    outi_ref[0:1, :] = (ew + 0.5).astype(jnp.int32)
    outi_ref[1:2, :] = (sw + 0.5).astype(jnp.int32)


def kernel(pos, dir, intensity, normals, offsets, reflectivity,
           map_to_element, map_to_surface):
    n = pos.shape[0]
    block = _BLOCK if n % _BLOCK == 0 else n
    grid = n // block

    pd = jnp.concatenate([
        pos.T, dir.T, intensity[None, :],
        jnp.zeros((1, n), jnp.float32),
    ], axis=0)

    s = normals.shape[0]
    a2 = jnp.zeros((2 * s, 8), jnp.float32)
    a2 = a2.at[0:s, 0:3].set(normals)
    a2 = a2.at[s:2 * s, 3:6].set(normals)
    a2 = a2.at[0:s, 7].set(offsets)

    tab = jnp.concatenate([
        normals.T,
        offsets[None, :],
        reflectivity[None, :],
        map_to_element[None, :].astype(jnp.float32),
        map_to_surface[None, :].astype(jnp.float32),
        jnp.zeros((1, s), jnp.float32),
    ], axis=0)

    outf, outi = pl.pallas_call(
        _scene_body,
        grid=(grid,),
        in_specs=[
            pl.BlockSpec((8, block), lambda i: (0, i)),
            pl.BlockSpec((2 * _S, 8), lambda i: (0, 0)),
            pl.BlockSpec((8, _S), lambda i: (0, 0)),
        ],
        out_specs=(
            pl.BlockSpec((8, block), lambda i: (0, i)),
            pl.BlockSpec((2, block), lambda i: (0, i)),
        ),
        out_shape=(
            jax.ShapeDtypeStruct((8, n), jnp.float32),
            jax.ShapeDtypeStruct((2, n), jnp.int32),
        ),
        compiler_params=pltpu.CompilerParams(
            dimension_semantics=("parallel",)),
    )(pd, a2, tab)

    next_pos = outf[0:3, :].T
    next_dir = outf[3:6, :].T
    next_intensity = outf[6, :]
    return (next_pos, next_dir, next_intensity, outi[0, :], outi[1, :])
